# baseline (speedup 1.0000x reference)
import functools

import jax
import jax.numpy as jnp
from jax import lax
from jax.experimental import pallas as pl
from jax.experimental.pallas import tpu as pltpu
from jax.experimental.pallas import tpu_sc as plsc

N = 10000
H = 128
E = 320000
NUM_LAYERS = 4
EPS = 1e-5

HH = H // 2
CK = 128
CH = 160
E_PAD = 16 * CH * CK
S_ROWS = 10240
RPT = S_ROWS // 16
DUMP = N
NBUF = 2
G = 2
NS = CH // G
GK = G * CK

BR = 2000


def _dg(a, b):
  return lax.dot_general(a, b, (((1,), (1,)), ((), ())),
                         preferred_element_type=jnp.float32)


def _ln_relu(t, g, b):
  m = jnp.mean(t, axis=-1, keepdims=True)
  v = jnp.mean((t - m) ** 2, axis=-1, keepdims=True)
  return jnp.maximum((t - m) * lax.rsqrt(v + EPS) * g + b, 0.0)


def _sigmoid(z):
  return 1.0 / (1.0 + jnp.exp(-z))


@functools.lru_cache(maxsize=2)
def _build_sc(with_cnt):
  out_type = [jax.ShapeDtypeStruct((2, S_ROWS, HH), jnp.float32)]
  scratch = [
      pltpu.VMEM((NS, GK), jnp.int32),
      pltpu.VMEM((NS, GK), jnp.int32),
      [pltpu.VMEM((GK, HH), jnp.float32) for _ in range(NBUF)],
      pltpu.VMEM_SHARED((S_ROWS, HH), jnp.float32),
      [pltpu.SemaphoreType.DMA for _ in range(NBUF)],
      [pltpu.SemaphoreType.DMA for _ in range(NBUF)],
  ]
  if with_cnt:
    out_type.append(jax.ShapeDtypeStruct((S_ROWS, 16), jnp.float32))
    scratch += [
        pltpu.VMEM((GK, 16), jnp.float32),
        pltpu.VMEM_SHARED((S_ROWS, 16), jnp.float32),
    ]

  mesh = plsc.VectorSubcoreMesh(core_axis_name="c", subcore_axis_name="s")

  def body(*refs):
    if with_cnt:
      (xs_hbm, ridx_hbm, cidx_hbm, zblk_hbm, ones_hbm, zcnt_hbm,
       s_out, cnt_out,
       ridx_v, cidx_v, bufs, s_sh, gsem, ssem,
       ones_v, cnt_sh) = refs
    else:
      (xs_hbm, ridx_hbm, cidx_hbm, zblk_hbm,
       s_out,
       ridx_v, cidx_v, bufs, s_sh, gsem, ssem) = refs

    c = lax.axis_index("c")
    s = lax.axis_index("s")

    pltpu.sync_copy(ridx_hbm.at[s], ridx_v)
    pltpu.sync_copy(cidx_hbm.at[s], cidx_v)
    pltpu.sync_copy(zblk_hbm, s_sh.at[pl.ds(s * RPT, RPT)])
    if with_cnt:
      pltpu.sync_copy(ones_hbm, ones_v)
      pltpu.sync_copy(zcnt_hbm, cnt_sh.at[pl.ds(s * RPT, RPT)])
    plsc.subcore_barrier()

    xh = xs_hbm.at[c]
    cslc = lambda j: cidx_v.at[j]
    rslc = lambda j: ridx_v.at[j]

    def drain_stream(j, b):
      pltpu.make_async_copy(xh.at[cslc(j)], bufs[b], gsem[b]).wait()
      pltpu.async_copy(bufs[b], s_sh.at[rslc(j)], ssem[b], add=True)
      if with_cnt:
        @pl.when(c == 0)
        def _():
          pltpu.sync_copy(ones_v, cnt_sh.at[rslc(j)], add=True)

    def wait_scatter(j, b):
      pltpu.make_async_copy(bufs[b], s_sh.at[rslc(j)], ssem[b]).wait()

    for b in range(NBUF):
      pltpu.async_copy(xh.at[cslc(b)], bufs[b], gsem[b])

    def step(j2, carry):
      for b in range(NBUF):
        j = j2 * NBUF + b
        drain_stream(j, b)
        wait_scatter(j, b)
        pltpu.async_copy(xh.at[cslc(j + NBUF)], bufs[b], gsem[b])
      return carry

    lax.fori_loop(0, NS // NBUF - 1, step, 0)
    for b in range(NBUF):
      j = NS - NBUF + b
      drain_stream(j, b)
      wait_scatter(j, b)

    plsc.subcore_barrier()
    pltpu.sync_copy(s_sh.at[pl.ds(s * RPT, RPT)],
                    s_out.at[c, pl.ds(s * RPT, RPT)])
    if with_cnt:
      @pl.when(c == 0)
      def _():
        pltpu.sync_copy(cnt_sh.at[pl.ds(s * RPT, RPT)],
                        cnt_out.at[pl.ds(s * RPT, RPT)])

  return pl.kernel(body,
                   out_type=tuple(out_type) if with_cnt else out_type[0],
                   mesh=mesh,
                   scratch_types=scratch,
                   compiler_params=pltpu.CompilerParams(
                       use_tc_tiling_on_sc=False))


def _pre_body(nf, qf, qe_w, qe_b, qe_g, qe_be, ntw1, ntw2, nt_b, nt_g, nt_be,
              wv, bv, ao_w, ao_b, o):
  q = _ln_relu(_dg(qf[...], qe_w[...]) + qe_b[...], qe_g[...], qe_be[...])
  t = _dg(nf[...], ntw1[...]) + _dg(q, ntw2[...]) + nt_b[...]
  x = _ln_relu(t, nt_g[...], nt_be[...])
  add_row = _dg(_dg(q, wv[...]) + bv[...], ao_w[...]) + ao_b[...]
  o[...] = x + add_row


def _layer_body(x_ref, sp, cp, wa, wb, mb, g, be, o):
  x = x_ref[...]
  ssum = jnp.concatenate([sp[0], sp[1]], axis=-1)
  cnt = cp[:, 0:1]
  inv = 1.0 / jnp.maximum(cnt, 1.0)
  t = _dg(x, wa[...]) + _dg(ssum * inv, wb[...]) + mb[...]
  agg = jnp.where(cnt > 0.0, t, x)
  o[...] = _ln_relu(agg, g[...], be[...])


def _heads_body(x_ref, mk_ref, v1a, v1b, v2w, v2b, v3, v3b,
                ema, emb, w32, p1a, p1b, p2w, p2b, p3, p3b, nv, pol):
  x = x_ref[...]
  mk = mk_ref[...]
  t = jnp.maximum(_dg(x, v1a[...]) + v1b[...], 0.0)
  t = jnp.maximum(_dg(t, v2w[...]) + v2b[...], 0.0)
  zv = jnp.sum(t * v3[...], axis=1, keepdims=True) + v3b[...]
  nv[...] = _sigmoid(zv)
  mp = mk * ema[...] + emb[...]
  u = jnp.maximum(_dg(x, p1a[...]) - _dg(mp, w32[...]) + p1b[...], 0.0)
  u = jnp.maximum(_dg(u, p2w[...]) + p2b[...], 0.0)
  zp = jnp.sum(u * p3[...], axis=1, keepdims=True) + p3b[...]
  pol[...] = _sigmoid(zp) * (1.0 - mk)


def _full(shape):
  nd = len(shape)
  return pl.BlockSpec(shape, lambda i: (0,) * nd)


def _pre_tc(nf, qf, *ws):
  grid = (N // BR,)
  in_specs = ([pl.BlockSpec((BR, H), lambda i: (i, 0)), _full(qf.shape)]
              + [_full(w.shape) for w in ws])
  return pl.pallas_call(
      _pre_body, grid=grid, in_specs=in_specs,
      out_specs=pl.BlockSpec((BR, H), lambda i: (i, 0)),
      out_shape=jax.ShapeDtypeStruct((N, H), jnp.float32),
  )(nf, qf, *ws)


def _layer_tc(x, sp, cp, *ws):
  grid = (N // BR,)
  in_specs = ([pl.BlockSpec((BR, H), lambda i: (i, 0)),
               pl.BlockSpec((2, BR, HH), lambda i: (0, i, 0)),
               pl.BlockSpec((BR, 16), lambda i: (i, 0))]
              + [_full(w.shape) for w in ws])
  return pl.pallas_call(
      _layer_body, grid=grid, in_specs=in_specs,
      out_specs=pl.BlockSpec((BR, H), lambda i: (i, 0)),
      out_shape=jax.ShapeDtypeStruct((N, H), jnp.float32),
  )(x, sp, cp, *ws)


def _heads_tc(x, mk, *ws):
  grid = (N // BR,)
  in_specs = ([pl.BlockSpec((BR, H), lambda i: (i, 0)),
               pl.BlockSpec((BR, 1), lambda i: (i, 0))]
              + [_full(w.shape) for w in ws])
  out_specs = [pl.BlockSpec((BR, 1), lambda i: (i, 0))] * 2
  return pl.pallas_call(
      _heads_body, grid=grid, in_specs=in_specs, out_specs=out_specs,
      out_shape=[jax.ShapeDtypeStruct((N, 1), jnp.float32)] * 2,
  )(x, mk, *ws)


def kernel(node_features, edge_index, question_features, expansion_mask,
           params):
  p = params
  f32 = jnp.float32
  r2 = lambda a: a.reshape(1, -1).astype(f32)

  row = edge_index[0].astype(jnp.int32)
  col = edge_index[1].astype(jnp.int32)
  pad = E_PAD - E
  ridx = jnp.concatenate([row, jnp.full((pad,), DUMP, jnp.int32)]
                         ).reshape(16, NS, GK)
  cidx = jnp.concatenate([col, jnp.zeros((pad,), jnp.int32)]
                         ).reshape(16, NS, GK)
  zblk = jnp.zeros((RPT, HH), f32)
  ones16 = jnp.ones((GK, 16), f32)
  zcnt = jnp.zeros((RPT, 16), f32)

  x = _pre_tc(node_features.astype(f32), question_features.astype(f32),
              p['qe_w'], r2(p['qe_b']), r2(p['qe_g']), r2(p['qe_be']),
              p['nt_w'][:, :H], p['nt_w'][:, H:], r2(p['nt_b']),
              r2(p['nt_g']), r2(p['nt_be']),
              p['in_w'][2 * H:3 * H], r2(p['in_b'][2 * H:3 * H]),
              p['ao_w'], r2(p['ao_b']))

  cp = None
  for l in range(NUM_LAYERS):
    xs = jnp.stack([x[:, :HH], x[:, HH:]])
    if l == 0:
      sp, cp = _build_sc(True)(xs, ridx, cidx, zblk, ones16, zcnt)
    else:
      sp = _build_sc(False)(xs, ridx, cidx, zblk)
    x = _layer_tc(x, sp, cp,
                  p['msg_w'][l][:, :H], p['msg_w'][l][:, H:2 * H],
                  r2(p['msg_b'][l]), r2(p['nu_g'][l]), r2(p['nu_be'][l]))

  w32 = (p['p1_w'][:, :H] + p['p1_w'][:, H:])[:, :H // 4]
  nv, pol = _heads_tc(
      x, expansion_mask.reshape(N, 1).astype(f32),
      p['v1_w'][:, :H], r2(p['v1_b']), p['v2_w'], r2(p['v2_b']),
      p['v3_w'], p['v3_b'].reshape(1, 1),
      p['em_w'].reshape(1, H // 4), r2(p['em_b']), w32,
      p['p1_w'][:, :H], r2(p['p1_b']), p['p2_w'], r2(p['p2_b']),
      p['p3_w'], p['p3_b'].reshape(1, 1))
  return nv, pol


# speedup vs baseline: 1.1669x; 1.1669x over previous
import functools

import jax
import jax.numpy as jnp
from jax import lax
from jax.experimental import pallas as pl
from jax.experimental.pallas import tpu as pltpu
from jax.experimental.pallas import tpu_sc as plsc

N = 10000
H = 128
E = 320000
NUM_LAYERS = 4
EPS = 1e-5

E_PAD = 327680
GK = 256
NS = E_PAD // (32 * GK)
S_ROWS = 10240
RPT = S_ROWS // 16
DUMP = N
NBUF = 2

BR = 2000


def _dg(a, b):
  return lax.dot_general(a, b, (((1,), (1,)), ((), ())),
                         preferred_element_type=jnp.float32)


def _ln_relu(t, g, b):
  m = jnp.mean(t, axis=-1, keepdims=True)
  v = jnp.mean((t - m) ** 2, axis=-1, keepdims=True)
  return jnp.maximum((t - m) * lax.rsqrt(v + EPS) * g + b, 0.0)


def _sigmoid(z):
  return 1.0 / (1.0 + jnp.exp(-z))


@functools.lru_cache(maxsize=2)
def _build_sc(with_cnt):
  out_type = [jax.ShapeDtypeStruct((2, S_ROWS, H), jnp.bfloat16)]
  scratch = [
      pltpu.VMEM((NS, GK), jnp.int32),
      pltpu.VMEM((NS, GK), jnp.int32),
      [pltpu.VMEM((GK, H), jnp.bfloat16) for _ in range(NBUF)],
      pltpu.VMEM_SHARED((S_ROWS, H), jnp.bfloat16),
      [pltpu.SemaphoreType.DMA for _ in range(NBUF)],
      [pltpu.SemaphoreType.DMA for _ in range(NBUF)],
  ]
  if with_cnt:
    out_type.append(jax.ShapeDtypeStruct((2, S_ROWS, 16), jnp.float32))
    scratch += [
        pltpu.VMEM((GK, 16), jnp.float32),
        pltpu.VMEM_SHARED((S_ROWS, 16), jnp.float32),
    ]

  mesh = plsc.VectorSubcoreMesh(core_axis_name="c", subcore_axis_name="s")

  def body(*refs):
    if with_cnt:
      (xs_hbm, ridx_hbm, cidx_hbm, zblk_hbm, ones_hbm, zcnt_hbm,
       s_out, cnt_out,
       ridx_v, cidx_v, bufs, s_sh, gsem, ssem,
       ones_v, cnt_sh) = refs
    else:
      (xs_hbm, ridx_hbm, cidx_hbm, zblk_hbm,
       s_out,
       ridx_v, cidx_v, bufs, s_sh, gsem, ssem) = refs

    c = lax.axis_index("c")
    s = lax.axis_index("s")
    wid = c * 16 + s

    pltpu.sync_copy(ridx_hbm.at[wid], ridx_v)
    pltpu.sync_copy(cidx_hbm.at[wid], cidx_v)
    pltpu.sync_copy(zblk_hbm, s_sh.at[pl.ds(s * RPT, RPT)])
    if with_cnt:
      pltpu.sync_copy(ones_hbm, ones_v)
      pltpu.sync_copy(zcnt_hbm, cnt_sh.at[pl.ds(s * RPT, RPT)])
    plsc.subcore_barrier()

    cslc = lambda j: cidx_v.at[j]
    rslc = lambda j: ridx_v.at[j]

    def drain_stream(j, b):
      pltpu.make_async_copy(xs_hbm.at[cslc(j)], bufs[b], gsem[b]).wait()
      pltpu.async_copy(bufs[b], s_sh.at[rslc(j)], ssem[b], add=True)
      if with_cnt:
        pltpu.sync_copy(ones_v, cnt_sh.at[rslc(j)], add=True)

    def wait_scatter(j, b):
      pltpu.make_async_copy(bufs[b], s_sh.at[rslc(j)], ssem[b]).wait()

    for b in range(NBUF):
      pltpu.async_copy(xs_hbm.at[cslc(b)], bufs[b], gsem[b])

    def step(j2, carry):
      for b in range(NBUF):
        j = j2 * NBUF + b
        drain_stream(j, b)
        wait_scatter(j, b)
        pltpu.async_copy(xs_hbm.at[cslc(j + NBUF)], bufs[b], gsem[b])
      return carry

    lax.fori_loop(0, NS // NBUF - 1, step, 0)
    for b in range(NBUF):
      j = NS - NBUF + b
      drain_stream(j, b)
      wait_scatter(j, b)

    plsc.subcore_barrier()
    pltpu.sync_copy(s_sh.at[pl.ds(s * RPT, RPT)],
                    s_out.at[c, pl.ds(s * RPT, RPT)])
    if with_cnt:
      pltpu.sync_copy(cnt_sh.at[pl.ds(s * RPT, RPT)],
                      cnt_out.at[c, pl.ds(s * RPT, RPT)])

  return pl.kernel(body,
                   out_type=tuple(out_type) if with_cnt else out_type[0],
                   mesh=mesh,
                   scratch_types=scratch,
                   compiler_params=pltpu.CompilerParams(
                       use_tc_tiling_on_sc=False))


def _pre_body(nf, qf, qe_w, qe_b, qe_g, qe_be, ntw1, ntw2, nt_b, nt_g, nt_be,
              wv, bv, ao_w, ao_b, o):
  q = _ln_relu(_dg(qf[...], qe_w[...]) + qe_b[...], qe_g[...], qe_be[...])
  t = _dg(nf[...], ntw1[...]) + _dg(q, ntw2[...]) + nt_b[...]
  x = _ln_relu(t, nt_g[...], nt_be[...])
  add_row = _dg(_dg(q, wv[...]) + bv[...], ao_w[...]) + ao_b[...]
  o[...] = x + add_row


def _layer_body(x_ref, sp, cp, wa, wb, mb, g, be, o):
  x = x_ref[...]
  ssum = sp[0].astype(jnp.float32) + sp[1].astype(jnp.float32)
  cnt = cp[0, :, 0:1] + cp[1, :, 0:1]
  inv = 1.0 / jnp.maximum(cnt, 1.0)
  t = _dg(x, wa[...]) + _dg(ssum * inv, wb[...]) + mb[...]
  agg = jnp.where(cnt > 0.0, t, x)
  o[...] = _ln_relu(agg, g[...], be[...])


def _heads_body(x_ref, mk_ref, v1a, v1b, v2w, v2b, v3, v3b,
                ema, emb, w32, p1a, p1b, p2w, p2b, p3, p3b, nv, pol):
  x = x_ref[...]
  mk = mk_ref[...]
  t = jnp.maximum(_dg(x, v1a[...]) + v1b[...], 0.0)
  t = jnp.maximum(_dg(t, v2w[...]) + v2b[...], 0.0)
  zv = jnp.sum(t * v3[...], axis=1, keepdims=True) + v3b[...]
  nv[...] = _sigmoid(zv)
  mp = mk * ema[...] + emb[...]
  u = jnp.maximum(_dg(x, p1a[...]) - _dg(mp, w32[...]) + p1b[...], 0.0)
  u = jnp.maximum(_dg(u, p2w[...]) + p2b[...], 0.0)
  zp = jnp.sum(u * p3[...], axis=1, keepdims=True) + p3b[...]
  pol[...] = _sigmoid(zp) * (1.0 - mk)


def _full(shape):
  nd = len(shape)
  return pl.BlockSpec(shape, lambda i: (0,) * nd)


def _pre_tc(nf, qf, *ws):
  grid = (N // BR,)
  in_specs = ([pl.BlockSpec((BR, H), lambda i: (i, 0)), _full(qf.shape)]
              + [_full(w.shape) for w in ws])
  return pl.pallas_call(
      _pre_body, grid=grid, in_specs=in_specs,
      out_specs=pl.BlockSpec((BR, H), lambda i: (i, 0)),
      out_shape=jax.ShapeDtypeStruct((N, H), jnp.float32),
  )(nf, qf, *ws)


def _layer_tc(x, sp, cp, *ws):
  grid = (N // BR,)
  in_specs = ([pl.BlockSpec((BR, H), lambda i: (i, 0)),
               pl.BlockSpec((2, BR, H), lambda i: (0, i, 0)),
               pl.BlockSpec((2, BR, 16), lambda i: (0, i, 0))]
              + [_full(w.shape) for w in ws])
  return pl.pallas_call(
      _layer_body, grid=grid, in_specs=in_specs,
      out_specs=pl.BlockSpec((BR, H), lambda i: (i, 0)),
      out_shape=jax.ShapeDtypeStruct((N, H), jnp.float32),
  )(x, sp, cp, *ws)


def _heads_tc(x, mk, *ws):
  grid = (N // BR,)
  in_specs = ([pl.BlockSpec((BR, H), lambda i: (i, 0)),
               pl.BlockSpec((BR, 1), lambda i: (i, 0))]
              + [_full(w.shape) for w in ws])
  out_specs = [pl.BlockSpec((BR, 1), lambda i: (i, 0))] * 2
  return pl.pallas_call(
      _heads_body, grid=grid, in_specs=in_specs, out_specs=out_specs,
      out_shape=[jax.ShapeDtypeStruct((N, 1), jnp.float32)] * 2,
  )(x, mk, *ws)


def kernel(node_features, edge_index, question_features, expansion_mask,
           params):
  p = params
  f32 = jnp.float32
  r2 = lambda a: a.reshape(1, -1).astype(f32)

  row = edge_index[0].astype(jnp.int32)
  col = edge_index[1].astype(jnp.int32)
  pad = E_PAD - E
  ridx = jnp.concatenate([row, jnp.full((pad,), DUMP, jnp.int32)]
                         ).reshape(32, NS, GK)
  cidx = jnp.concatenate([col, jnp.zeros((pad,), jnp.int32)]
                         ).reshape(32, NS, GK)
  zblk = jnp.zeros((RPT, H), jnp.bfloat16)
  ones16 = jnp.ones((GK, 16), f32)
  zcnt = jnp.zeros((RPT, 16), f32)

  x = _pre_tc(node_features.astype(f32), question_features.astype(f32),
              p['qe_w'], r2(p['qe_b']), r2(p['qe_g']), r2(p['qe_be']),
              p['nt_w'][:, :H], p['nt_w'][:, H:], r2(p['nt_b']),
              r2(p['nt_g']), r2(p['nt_be']),
              p['in_w'][2 * H:3 * H], r2(p['in_b'][2 * H:3 * H]),
              p['ao_w'], r2(p['ao_b']))

  cp = None
  for l in range(NUM_LAYERS):
    xs = x.astype(jnp.bfloat16)
    if l == 0:
      sp, cp = _build_sc(True)(xs, ridx, cidx, zblk, ones16, zcnt)
    else:
      sp = _build_sc(False)(xs, ridx, cidx, zblk)
    x = _layer_tc(x, sp, cp,
                  p['msg_w'][l][:, :H], p['msg_w'][l][:, H:2 * H],
                  r2(p['msg_b'][l]), r2(p['nu_g'][l]), r2(p['nu_be'][l]))

  w32 = (p['p1_w'][:, :H] + p['p1_w'][:, H:])[:, :H // 4]
  nv, pol = _heads_tc(
      x, expansion_mask.reshape(N, 1).astype(f32),
      p['v1_w'][:, :H], r2(p['v1_b']), p['v2_w'], r2(p['v2_b']),
      p['v3_w'], p['v3_b'].reshape(1, 1),
      p['em_w'].reshape(1, H // 4), r2(p['em_b']), w32,
      p['p1_w'][:, :H], r2(p['p1_b']), p['p2_w'], r2(p['p2_b']),
      p['p3_w'], p['p3_b'].reshape(1, 1))
  return nv, pol


# speedup vs baseline: 1.1672x; 1.0002x over previous
import functools

import jax
import jax.numpy as jnp
from jax import lax
from jax.experimental import pallas as pl
from jax.experimental.pallas import tpu as pltpu
from jax.experimental.pallas import tpu_sc as plsc

N = 10000
H = 128
E = 320000
NUM_LAYERS = 4
EPS = 1e-5

E_PAD = 327680
GK = 256
NS = E_PAD // (32 * GK)
S_ROWS = 10240
RPT = S_ROWS // 16
DUMP = N
NBUF = 2

BR = 2000


def _dg(a, b):
  return lax.dot_general(a, b, (((1,), (1,)), ((), ())),
                         preferred_element_type=jnp.float32)


def _ln_relu(t, g, b):
  m = jnp.mean(t, axis=-1, keepdims=True)
  v = jnp.mean((t - m) ** 2, axis=-1, keepdims=True)
  return jnp.maximum((t - m) * lax.rsqrt(v + EPS) * g + b, 0.0)


def _sigmoid(z):
  return 1.0 / (1.0 + jnp.exp(-z))


@functools.lru_cache(maxsize=2)
def _build_sc(with_cnt):
  out_type = [jax.ShapeDtypeStruct((2, S_ROWS, H), jnp.bfloat16)]
  scratch = [
      pltpu.VMEM((NS, GK), jnp.int32),
      pltpu.VMEM((NS, GK), jnp.int32),
      [pltpu.VMEM((GK, H), jnp.bfloat16) for _ in range(NBUF)],
      pltpu.VMEM_SHARED((S_ROWS, H), jnp.bfloat16),
      [pltpu.SemaphoreType.DMA for _ in range(NBUF)],
      [pltpu.SemaphoreType.DMA for _ in range(NBUF)],
  ]
  if with_cnt:
    out_type.append(jax.ShapeDtypeStruct((2, S_ROWS, 16), jnp.float32))
    scratch += [
        pltpu.VMEM((GK, 16), jnp.float32),
        pltpu.VMEM_SHARED((S_ROWS, 16), jnp.float32),
    ]

  mesh = plsc.VectorSubcoreMesh(core_axis_name="c", subcore_axis_name="s")

  def body(*refs):
    if with_cnt:
      (xs_hbm, ridx_hbm, cidx_hbm, zblk_hbm, ones_hbm, zcnt_hbm,
       s_out, cnt_out,
       ridx_v, cidx_v, bufs, s_sh, gsem, ssem,
       ones_v, cnt_sh) = refs
    else:
      (xs_hbm, ridx_hbm, cidx_hbm, zblk_hbm,
       s_out,
       ridx_v, cidx_v, bufs, s_sh, gsem, ssem) = refs

    c = lax.axis_index("c")
    s = lax.axis_index("s")
    wid = s * 2 + c

    pltpu.sync_copy(ridx_hbm.at[wid], ridx_v)
    pltpu.sync_copy(cidx_hbm.at[wid], cidx_v)
    pltpu.sync_copy(zblk_hbm, s_sh.at[pl.ds(s * RPT, RPT)])
    if with_cnt:
      pltpu.sync_copy(ones_hbm, ones_v)
      pltpu.sync_copy(zcnt_hbm, cnt_sh.at[pl.ds(s * RPT, RPT)])
    plsc.subcore_barrier()

    cslc = lambda j: cidx_v.at[j]
    rslc = lambda j: ridx_v.at[j]

    def drain_stream(j, b):
      pltpu.make_async_copy(xs_hbm.at[cslc(j)], bufs[b], gsem[b]).wait()
      pltpu.async_copy(bufs[b], s_sh.at[rslc(j)], ssem[b], add=True)
      if with_cnt:
        pltpu.sync_copy(ones_v, cnt_sh.at[rslc(j)], add=True)

    def wait_scatter(j, b):
      pltpu.make_async_copy(bufs[b], s_sh.at[rslc(j)], ssem[b]).wait()

    for b in range(NBUF):
      pltpu.async_copy(xs_hbm.at[cslc(b)], bufs[b], gsem[b])

    def step(j2, carry):
      for b in range(NBUF):
        j = j2 * NBUF + b
        drain_stream(j, b)
        wait_scatter(j, b)
        pltpu.async_copy(xs_hbm.at[cslc(j + NBUF)], bufs[b], gsem[b])
      return carry

    lax.fori_loop(0, NS // NBUF - 1, step, 0)
    for b in range(NBUF):
      j = NS - NBUF + b
      drain_stream(j, b)
      wait_scatter(j, b)

    plsc.subcore_barrier()
    pltpu.sync_copy(s_sh.at[pl.ds(s * RPT, RPT)],
                    s_out.at[c, pl.ds(s * RPT, RPT)])
    if with_cnt:
      pltpu.sync_copy(cnt_sh.at[pl.ds(s * RPT, RPT)],
                      cnt_out.at[c, pl.ds(s * RPT, RPT)])

  return pl.kernel(body,
                   out_type=tuple(out_type) if with_cnt else out_type[0],
                   mesh=mesh,
                   scratch_types=scratch,
                   compiler_params=pltpu.CompilerParams(
                       use_tc_tiling_on_sc=False))


def _pre_body(nf, qf, qe_w, qe_b, qe_g, qe_be, ntw1, ntw2, nt_b, nt_g, nt_be,
              wv, bv, ao_w, ao_b, o):
  q = _ln_relu(_dg(qf[...], qe_w[...]) + qe_b[...], qe_g[...], qe_be[...])
  t = _dg(nf[...], ntw1[...]) + _dg(q, ntw2[...]) + nt_b[...]
  x = _ln_relu(t, nt_g[...], nt_be[...])
  add_row = _dg(_dg(q, wv[...]) + bv[...], ao_w[...]) + ao_b[...]
  o[...] = x + add_row


def _layer_body(x_ref, sp, cp, wa, wb, mb, g, be, o):
  x = x_ref[...]
  ssum = sp[0].astype(jnp.float32) + sp[1].astype(jnp.float32)
  cnt = cp[0, :, 0:1] + cp[1, :, 0:1]
  inv = 1.0 / jnp.maximum(cnt, 1.0)
  t = _dg(x, wa[...]) + _dg(ssum * inv, wb[...]) + mb[...]
  agg = jnp.where(cnt > 0.0, t, x)
  o[...] = _ln_relu(agg, g[...], be[...])


def _heads_body(x_ref, mk_ref, v1a, v1b, v2w, v2b, v3, v3b,
                ema, emb, w32, p1a, p1b, p2w, p2b, p3, p3b, nv, pol):
  x = x_ref[...]
  mk = mk_ref[...]
  t = jnp.maximum(_dg(x, v1a[...]) + v1b[...], 0.0)
  t = jnp.maximum(_dg(t, v2w[...]) + v2b[...], 0.0)
  zv = jnp.sum(t * v3[...], axis=1, keepdims=True) + v3b[...]
  nv[...] = _sigmoid(zv)
  mp = mk * ema[...] + emb[...]
  u = jnp.maximum(_dg(x, p1a[...]) - _dg(mp, w32[...]) + p1b[...], 0.0)
  u = jnp.maximum(_dg(u, p2w[...]) + p2b[...], 0.0)
  zp = jnp.sum(u * p3[...], axis=1, keepdims=True) + p3b[...]
  pol[...] = _sigmoid(zp) * (1.0 - mk)


def _full(shape):
  nd = len(shape)
  return pl.BlockSpec(shape, lambda i: (0,) * nd)


def _pre_tc(nf, qf, *ws):
  grid = (N // BR,)
  in_specs = ([pl.BlockSpec((BR, H), lambda i: (i, 0)), _full(qf.shape)]
              + [_full(w.shape) for w in ws])
  return pl.pallas_call(
      _pre_body, grid=grid, in_specs=in_specs,
      out_specs=pl.BlockSpec((BR, H), lambda i: (i, 0)),
      out_shape=jax.ShapeDtypeStruct((N, H), jnp.float32),
  )(nf, qf, *ws)


def _layer_tc(x, sp, cp, *ws):
  grid = (N // BR,)
  in_specs = ([pl.BlockSpec((BR, H), lambda i: (i, 0)),
               pl.BlockSpec((2, BR, H), lambda i: (0, i, 0)),
               pl.BlockSpec((2, BR, 16), lambda i: (0, i, 0))]
              + [_full(w.shape) for w in ws])
  return pl.pallas_call(
      _layer_body, grid=grid, in_specs=in_specs,
      out_specs=pl.BlockSpec((BR, H), lambda i: (i, 0)),
      out_shape=jax.ShapeDtypeStruct((N, H), jnp.float32),
  )(x, sp, cp, *ws)


def _heads_tc(x, mk, *ws):
  grid = (N // BR,)
  in_specs = ([pl.BlockSpec((BR, H), lambda i: (i, 0)),
               pl.BlockSpec((BR, 1), lambda i: (i, 0))]
              + [_full(w.shape) for w in ws])
  out_specs = [pl.BlockSpec((BR, 1), lambda i: (i, 0))] * 2
  return pl.pallas_call(
      _heads_body, grid=grid, in_specs=in_specs, out_specs=out_specs,
      out_shape=[jax.ShapeDtypeStruct((N, 1), jnp.float32)] * 2,
  )(x, mk, *ws)


def kernel(node_features, edge_index, question_features, expansion_mask,
           params):
  p = params
  f32 = jnp.float32
  r2 = lambda a: a.reshape(1, -1).astype(f32)

  row = edge_index[0].astype(jnp.int32)
  col = edge_index[1].astype(jnp.int32)
  pad = E_PAD - E
  ridx = jnp.concatenate([row, jnp.full((pad,), DUMP, jnp.int32)]
                         ).reshape(32, NS, GK)
  cidx = jnp.concatenate([col, jnp.zeros((pad,), jnp.int32)]
                         ).reshape(32, NS, GK)
  zblk = jnp.zeros((RPT, H), jnp.bfloat16)
  ones16 = jnp.ones((GK, 16), f32)
  zcnt = jnp.zeros((RPT, 16), f32)

  x = _pre_tc(node_features.astype(f32), question_features.astype(f32),
              p['qe_w'], r2(p['qe_b']), r2(p['qe_g']), r2(p['qe_be']),
              p['nt_w'][:, :H], p['nt_w'][:, H:], r2(p['nt_b']),
              r2(p['nt_g']), r2(p['nt_be']),
              p['in_w'][2 * H:3 * H], r2(p['in_b'][2 * H:3 * H]),
              p['ao_w'], r2(p['ao_b']))

  cp = None
  for l in range(NUM_LAYERS):
    xs = x.astype(jnp.bfloat16)
    if l == 0:
      sp, cp = _build_sc(True)(xs, ridx, cidx, zblk, ones16, zcnt)
    else:
      sp = _build_sc(False)(xs, ridx, cidx, zblk)
    x = _layer_tc(x, sp, cp,
                  p['msg_w'][l][:, :H], p['msg_w'][l][:, H:2 * H],
                  r2(p['msg_b'][l]), r2(p['nu_g'][l]), r2(p['nu_be'][l]))

  w32 = (p['p1_w'][:, :H] + p['p1_w'][:, H:])[:, :H // 4]
  nv, pol = _heads_tc(
      x, expansion_mask.reshape(N, 1).astype(f32),
      p['v1_w'][:, :H], r2(p['v1_b']), p['v2_w'], r2(p['v2_b']),
      p['v3_w'], p['v3_b'].reshape(1, 1),
      p['em_w'].reshape(1, H // 4), r2(p['em_b']), w32,
      p['p1_w'][:, :H], r2(p['p1_b']), p['p2_w'], r2(p['p2_b']),
      p['p3_w'], p['p3_b'].reshape(1, 1))
  return nv, pol
